# baseline (speedup 1.0000x reference)
import functools

import jax
import jax.numpy as jnp
import numpy as np
from jax import lax
from jax.experimental import pallas as pl
from jax.experimental.pallas import tpu as pltpu
from jax.experimental.pallas import tpu_sc as plsc

B = 128
T = 32768
NW = 32
ROWS_PER_W = B // NW
NSTEP = T // 16
INT_MIN = np.int32(-2147483648)
M31 = np.int32(0x7FFFFFFF)


def _splat(v):
    return jnp.broadcast_to(v, (16,))


def _pick(vec, lane_splat):
    dnums = lax.GatherDimensionNumbers(
        offset_dims=(), collapsed_slice_dims=(0,), start_index_map=(0,))
    return lax.gather(vec, lane_splat[:, None], dnums, (1,),
                      mode=lax.GatherScatterMode.PROMISE_IN_BOUNDS)


def _topk_mean_kernel(x_hbm, sl_hbm, out_hbm, vals_v, keys_v, hist_v,
                      tot_v, sl_v, res_v, sem):
    wid = lax.axis_index("s") * 2 + lax.axis_index("c")
    pltpu.sync_copy(sl_hbm, sl_v)

    lane = lax.iota(jnp.int32, 16)
    ones = jnp.ones((16,), jnp.int32)
    zf = jnp.zeros((16,), jnp.float32)
    zi = jnp.zeros((16,), jnp.int32)
    fifteen = _splat(np.int32(15))

    r0 = wid * ROWS_PER_W
    pltpu.async_copy(x_hbm.at[pl.ds(r0 * T, T)], vals_v, sem)

    def zero_hist():
        @plsc.parallel_loop(0, 256, unroll=8)
        def _(i):
            hist_v[pl.ds(i * 16, 16)] = zi

    def reduce_hist():
        @plsc.parallel_loop(0, 16, unroll=2)
        def _(c):
            bvec = (c * 16 + lane) * 16
            acc = zi
            for l in range(16):
                acc = acc + plsc.load_gather(hist_v, [bvec + l])
            tot_v[pl.ds(c * 16, 16)] = acc

    def scan_hist(kr):
        @plsc.parallel_loop(0, 16, carry=(zi, _splat(np.int32(-1)), ones))
        def carry(i, carry):
            above, bstar, krn = carry
            c = 15 - i
            tvec = tot_v[pl.ds(c * 16, 16)]
            cs = plsc.cumsum(tvec)
            ctot = _pick(cs, fifteen)
            s = above + ctot - cs + tvec
            mask = s >= kr
            nhit = plsc.all_reduce_population_count(mask)
            hit = (bstar < 0) & (nhit > 0)
            lane_star = nhit - 1
            cs_star = _pick(cs, jnp.maximum(lane_star, 0))
            cum_above = above + ctot - cs_star
            bstar = jnp.where(hit, c * 16 + lane_star, bstar)
            krn = jnp.where(hit, kr - cum_above, krn)
            return above + ctot, bstar, krn
        _, bstar, krn = carry
        return bstar, krn

    def row_body(j, res):
        r = r0 + j
        pltpu.make_async_copy(x_hbm.at[pl.ds(r * T, T)], vals_v, sem).wait()

        chunk = sl_v[pl.ds((r // 16) * 16, 16)]
        sl = _pick(chunk, _splat(r % 16))
        k = sl // 16 + 1
        thr = (sl - lane + 15) >> 4

        zero_hist()

        @plsc.parallel_loop(0, NSTEP, unroll=4)
        def _(i):
            v = vals_v[pl.ds(i * 16, 16)]
            m = plsc.bitcast(v, jnp.int32)
            m = jnp.where(m >= 0, m, m ^ M31)
            m = jnp.where(i < thr, m, INT_MIN)
            keys_v[pl.ds(i * 16, 16)] = m
            idx = (lax.shift_right_logical(m ^ INT_MIN, 20) & 0xFF0) | lane
            plsc.addupdate_scatter(hist_v, [idx], ones)

        @pl.when(j < ROWS_PER_W - 1)
        def _():
            pltpu.async_copy(x_hbm.at[pl.ds((r + 1) * T, T)], vals_v, sem)

        reduce_hist()
        bstar, kr = scan_hist(k)
        prefix = bstar << 24

        for lvl in range(1, 4):
            shift = 24 - 8 * lvl
            zero_hist()
            phi = lax.shift_right_logical(prefix, shift + 8)

            @plsc.parallel_loop(0, NSTEP, unroll=4)
            def _(i):
                uk = keys_v[pl.ds(i * 16, 16)] ^ INT_MIN
                selm = lax.shift_right_logical(uk, shift + 8) == phi
                idx = ((lax.shift_right_logical(uk, shift) & 255) << 4) | lane
                plsc.addupdate_scatter(hist_v, [idx], ones, mask=selm)

            reduce_hist()
            bstar, kr = scan_hist(kr)
            prefix = prefix | (bstar << shift)

        tau_m = prefix ^ INT_MIN
        tau_b = jnp.where(tau_m >= 0, tau_m, tau_m ^ M31)
        tau_f = plsc.bitcast(tau_b, jnp.float32)

        @plsc.parallel_loop(0, NSTEP, step=8, carry=(zf,) * 8)
        def accs(i, accs):
            out = []
            for t in range(8):
                m = keys_v[pl.ds((i + t) * 16, 16)]
                bb = jnp.where(m >= 0, m, m ^ M31)
                v = plsc.bitcast(bb, jnp.float32)
                out.append(accs[t] + jnp.where(m > tau_m, v, 0.0))
            return tuple(out)

        acc = accs[0]
        for t in range(1, 8):
            acc = acc + accs[t]
        sum_gt = _splat(jnp.sum(acc))
        rj = (sum_gt + kr.astype(jnp.float32) * tau_f) / k.astype(jnp.float32)
        return jnp.where(lane == j, rj, res)

    res = lax.fori_loop(0, ROWS_PER_W, row_body, zf)
    res_v[...] = res
    pltpu.sync_copy(res_v, out_hbm.at[wid])


@functools.partial(
    pl.kernel,
    out_type=jax.ShapeDtypeStruct((NW, 16), jnp.float32),
    mesh=plsc.VectorSubcoreMesh(core_axis_name="c", subcore_axis_name="s"),
    compiler_params=pltpu.CompilerParams(needs_layout_passes=False),
    scratch_types=[
        pltpu.VMEM((T,), jnp.float32),
        pltpu.VMEM((T,), jnp.int32),
        pltpu.VMEM((4096,), jnp.int32),
        pltpu.VMEM((256,), jnp.int32),
        pltpu.VMEM((B,), jnp.int32),
        pltpu.VMEM((16,), jnp.float32),
        pltpu.SemaphoreType.DMA,
    ],
)
def _topk_mean_call(x_hbm, sl_hbm, out_hbm, *scratch):
    _topk_mean_kernel(x_hbm, sl_hbm, out_hbm, *scratch)


def kernel(x, seq_len):
    xf = jnp.reshape(x, (B * T,)).astype(jnp.float32)
    slx = seq_len.astype(jnp.int32)
    out = _topk_mean_call(xf, slx)
    return jnp.reshape(out[:, :ROWS_PER_W], (B,))


# speedup vs baseline: 1.0380x; 1.0380x over previous
import functools

import jax
import jax.numpy as jnp
import numpy as np
from jax import lax
from jax.experimental import pallas as pl
from jax.experimental.pallas import tpu as pltpu
from jax.experimental.pallas import tpu_sc as plsc

B = 128
T = 32768
NW = 32
ROWS_PER_W = B // NW
NSTEP = T // 16
INT_MIN = np.int32(-2147483648)
M31 = np.int32(0x7FFFFFFF)


def _splat(v):
    return jnp.broadcast_to(v, (16,))


def _pick(vec, lane_splat):
    dnums = lax.GatherDimensionNumbers(
        offset_dims=(), collapsed_slice_dims=(0,), start_index_map=(0,))
    return lax.gather(vec, lane_splat[:, None], dnums, (1,),
                      mode=lax.GatherScatterMode.PROMISE_IN_BOUNDS)


def _topk_mean_kernel(x_hbm, sl_hbm, out_hbm, vals_v, keys_v, hist_v,
                      tot_v, sl_v, res_v, sem):
    wid = lax.axis_index("s") * 2 + lax.axis_index("c")
    pltpu.sync_copy(sl_hbm, sl_v)

    lane = lax.iota(jnp.int32, 16)
    ones = jnp.ones((16,), jnp.int32)
    zf = jnp.zeros((16,), jnp.float32)
    zi = jnp.zeros((16,), jnp.int32)
    fifteen = _splat(np.int32(15))

    r0 = wid * ROWS_PER_W
    pltpu.async_copy(x_hbm.at[pl.ds(r0 * T, T)], vals_v, sem)

    def zero_hist():
        @plsc.parallel_loop(0, 256, unroll=8)
        def _(i):
            hist_v[pl.ds(i * 16, 16)] = zi

    def reduce_hist():
        @plsc.parallel_loop(0, 16, unroll=2)
        def _(c):
            bvec = (c * 16 + lane) * 16
            acc = zi
            for l in range(16):
                acc = acc + plsc.load_gather(hist_v, [bvec + l])
            tot_v[pl.ds(c * 16, 16)] = acc

    def scan_hist(kr):
        @plsc.parallel_loop(0, 16, carry=(zi, _splat(np.int32(-1)), ones))
        def carry(i, carry):
            above, bstar, krn = carry
            c = 15 - i
            tvec = tot_v[pl.ds(c * 16, 16)]
            cs = plsc.cumsum(tvec)
            ctot = _pick(cs, fifteen)
            s = above + ctot - cs + tvec
            mask = s >= kr
            nhit = plsc.all_reduce_population_count(mask)
            hit = (bstar < 0) & (nhit > 0)
            lane_star = nhit - 1
            cs_star = _pick(cs, jnp.maximum(lane_star, 0))
            cum_above = above + ctot - cs_star
            bstar = jnp.where(hit, c * 16 + lane_star, bstar)
            krn = jnp.where(hit, kr - cum_above, krn)
            return above + ctot, bstar, krn
        _, bstar, krn = carry
        return bstar, krn

    def row_body(j, res):
        r = r0 + j
        pltpu.make_async_copy(x_hbm.at[pl.ds(r * T, T)], vals_v, sem).wait()

        chunk = sl_v[pl.ds((r // 16) * 16, 16)]
        sl = _pick(chunk, _splat(r % 16))
        k = sl // 16 + 1
        thr = (sl - lane + 15) >> 4

        zero_hist()

        @plsc.parallel_loop(0, NSTEP, unroll=8)
        def _(i):
            v = vals_v[pl.ds(i * 16, 16)]
            m = plsc.bitcast(v, jnp.int32)
            m = jnp.where(m >= 0, m, m ^ M31)
            m = jnp.where(i < thr, m, INT_MIN)
            keys_v[pl.ds(i * 16, 16)] = m
            idx = (lax.shift_right_logical(m ^ INT_MIN, 20) & 0xFF0) | lane
            plsc.addupdate_scatter(hist_v, [idx], ones)

        @pl.when(j < ROWS_PER_W - 1)
        def _():
            pltpu.async_copy(x_hbm.at[pl.ds((r + 1) * T, T)], vals_v, sem)

        reduce_hist()
        bstar, kr = scan_hist(k)
        prefix = bstar << 24

        for lvl in range(1, 4):
            shift = 24 - 8 * lvl
            zero_hist()
            phi = lax.shift_right_logical(prefix, shift + 8)

            @plsc.parallel_loop(0, NSTEP, unroll=8)
            def _(i):
                uk = keys_v[pl.ds(i * 16, 16)] ^ INT_MIN
                selm = lax.shift_right_logical(uk, shift + 8) == phi
                idx = ((lax.shift_right_logical(uk, shift) & 255) << 4) | lane
                plsc.addupdate_scatter(hist_v, [idx], ones, mask=selm)

            reduce_hist()
            bstar, kr = scan_hist(kr)
            prefix = prefix | (bstar << shift)

        tau_m = prefix ^ INT_MIN
        tau_b = jnp.where(tau_m >= 0, tau_m, tau_m ^ M31)
        tau_f = plsc.bitcast(tau_b, jnp.float32)

        @plsc.parallel_loop(0, NSTEP, step=8, carry=(zf,) * 8)
        def accs(i, accs):
            out = []
            for t in range(8):
                m = keys_v[pl.ds((i + t) * 16, 16)]
                bb = jnp.where(m >= 0, m, m ^ M31)
                v = plsc.bitcast(bb, jnp.float32)
                out.append(accs[t] + jnp.where(m > tau_m, v, 0.0))
            return tuple(out)

        acc = accs[0]
        for t in range(1, 8):
            acc = acc + accs[t]
        sum_gt = _splat(jnp.sum(acc))
        rj = (sum_gt + kr.astype(jnp.float32) * tau_f) / k.astype(jnp.float32)
        return jnp.where(lane == j, rj, res)

    res = lax.fori_loop(0, ROWS_PER_W, row_body, zf)
    res_v[...] = res
    pltpu.sync_copy(res_v, out_hbm.at[wid])


@functools.partial(
    pl.kernel,
    out_type=jax.ShapeDtypeStruct((NW, 16), jnp.float32),
    mesh=plsc.VectorSubcoreMesh(core_axis_name="c", subcore_axis_name="s"),
    compiler_params=pltpu.CompilerParams(needs_layout_passes=False),
    scratch_types=[
        pltpu.VMEM((T,), jnp.float32),
        pltpu.VMEM((T,), jnp.int32),
        pltpu.VMEM((4096,), jnp.int32),
        pltpu.VMEM((256,), jnp.int32),
        pltpu.VMEM((B,), jnp.int32),
        pltpu.VMEM((16,), jnp.float32),
        pltpu.SemaphoreType.DMA,
    ],
)
def _topk_mean_call(x_hbm, sl_hbm, out_hbm, *scratch):
    _topk_mean_kernel(x_hbm, sl_hbm, out_hbm, *scratch)


def kernel(x, seq_len):
    xf = jnp.reshape(x, (B * T,)).astype(jnp.float32)
    slx = seq_len.astype(jnp.int32)
    out = _topk_mean_call(xf, slx)
    return jnp.reshape(out[:, :ROWS_PER_W], (B,))
